# baseline (speedup 1.0000x reference)
import math

import jax
import jax.numpy as jnp
from jax.experimental import pallas as pl
from jax.experimental.pallas import tpu as pltpu

L = 200
K = 16
D = 32
TAIL0 = L - K
T2 = 2 * K
WSUM = float(sum(math.exp(i - L) for i in range(L)))
WT = [math.exp(t - K) for t in range(K)]


def _gae_kernel(srcT_ref, dstT_ref, tlT_ref, w1b_ref, b1b_ref, w2_ref,
                b2b_ref, so_ref, do_ref):
    src = srcT_ref[...]
    dst = dstT_ref[...]
    tl = tlT_ref[...]
    rb = src.shape[1]

    acc_s = jnp.zeros((T2, rb), jnp.int32)
    acc_d = jnp.zeros((T2, rb), jnp.int32)
    for j in range(L):
        cj_s = jnp.broadcast_to(src[j:j + 1, :], (T2, rb))
        acc_s = acc_s + (tl == cj_s).astype(jnp.int32)
        cj_d = jnp.broadcast_to(dst[j:j + 1, :], (T2, rb))
        acc_d = acc_d + (tl == cj_d).astype(jnp.int32)
    valid = tl != 0
    cs = jnp.where(valid, acc_s, 0).astype(jnp.float32)
    cd = jnp.where(valid, acc_d, 0).astype(jnp.float32)

    w1b = w1b_ref[...]
    b1b = b1b_ref[...]
    accf_s = jnp.zeros((D, rb), jnp.float32)
    accf_d = jnp.zeros((D, rb), jnp.float32)
    for t in range(K):
        wt = WT[t]
        c1 = jnp.broadcast_to(cs[t:t + 1, :], (D, rb))
        c2 = jnp.broadcast_to(cd[t:t + 1, :], (D, rb))
        accf_s = accf_s + wt * (jax.nn.relu(c1 * w1b + b1b)
                                + jax.nn.relu(c2 * w1b + b1b))
        c1d = jnp.broadcast_to(cs[K + t:K + t + 1, :], (D, rb))
        c2d = jnp.broadcast_to(cd[K + t:K + t + 1, :], (D, rb))
        accf_d = accf_d + wt * (jax.nn.relu(c1d * w1b + b1b)
                                + jax.nn.relu(c2d * w1b + b1b))

    w2 = w2_ref[...]
    b2b = b2b_ref[...]
    inv = 1.0 / WSUM
    so_ref[...] = (jnp.dot(w2, accf_s, preferred_element_type=jnp.float32)
                   * inv + b2b)
    do_ref[...] = (jnp.dot(w2, accf_d, preferred_element_type=jnp.float32)
                   * inv + b2b)


def kernel(src_padded_nodes_neighbor_ids, dst_padded_nodes_neighbor_ids,
           W1, b1, W2, b2):
    src = src_padded_nodes_neighbor_ids
    dst = dst_padded_nodes_neighbor_ids
    B = src.shape[0]
    f32 = jnp.float32

    srcT = src.T
    dstT = dst.T
    tlT = jnp.concatenate([src[:, TAIL0:], dst[:, TAIL0:]], axis=1).T

    ones = jnp.ones((1, B), f32)
    w1b = W1.reshape(D, 1) * ones
    b1b = b1.reshape(D, 1) * ones
    b2b = (2.0 * b2).reshape(D, 1) * ones

    Rb = 512
    grid = (B // Rb,)
    so, do = pl.pallas_call(
        _gae_kernel,
        grid=grid,
        in_specs=[
            pl.BlockSpec((L, Rb), lambda i: (0, i)),
            pl.BlockSpec((L, Rb), lambda i: (0, i)),
            pl.BlockSpec((T2, Rb), lambda i: (0, i)),
            pl.BlockSpec((D, Rb), lambda i: (0, i)),
            pl.BlockSpec((D, Rb), lambda i: (0, i)),
            pl.BlockSpec((D, D), lambda i: (0, 0)),
            pl.BlockSpec((D, Rb), lambda i: (0, i)),
        ],
        out_specs=[pl.BlockSpec((D, Rb), lambda i: (0, i)),
                   pl.BlockSpec((D, Rb), lambda i: (0, i))],
        out_shape=[jax.ShapeDtypeStruct((D, B), f32),
                   jax.ShapeDtypeStruct((D, B), f32)],
        compiler_params=pltpu.CompilerParams(
            dimension_semantics=("parallel",)),
    )(srcT, dstT, tlT, w1b, b1b, W2, b2b)
    return (so.T, do.T)


# speedup vs baseline: 1.0052x; 1.0052x over previous
import math

import jax
import jax.numpy as jnp
from jax.experimental import pallas as pl
from jax.experimental.pallas import tpu as pltpu

L = 200
K = 16
D = 32
TAIL0 = L - K
T2 = 2 * K
WSUM = float(sum(math.exp(i - L) for i in range(L)))
WT = [math.exp(t - K) for t in range(K)]


def _gae_kernel(srcT_ref, dstT_ref, tlT_ref, w1b_ref, b1b_ref, w2_ref,
                b2b_ref, so_ref, do_ref):
    src = srcT_ref[...]
    dst = dstT_ref[...]
    tl = tlT_ref[...]
    rb = src.shape[1]

    acc_s = jnp.zeros((T2, rb), jnp.int32)
    acc_d = jnp.zeros((T2, rb), jnp.int32)
    for j in range(L):
        cj_s = jnp.broadcast_to(src[j:j + 1, :], (T2, rb))
        acc_s = acc_s + (tl == cj_s).astype(jnp.int32)
        cj_d = jnp.broadcast_to(dst[j:j + 1, :], (T2, rb))
        acc_d = acc_d + (tl == cj_d).astype(jnp.int32)
    valid = tl != 0
    cs = jnp.where(valid, acc_s, 0).astype(jnp.float32)
    cd = jnp.where(valid, acc_d, 0).astype(jnp.float32)

    w1b = w1b_ref[...]
    b1b = b1b_ref[...]
    accf_s = jnp.zeros((D, rb), jnp.float32)
    accf_d = jnp.zeros((D, rb), jnp.float32)
    for t in range(K):
        wt = WT[t]
        c1 = jnp.broadcast_to(cs[t:t + 1, :], (D, rb))
        c2 = jnp.broadcast_to(cd[t:t + 1, :], (D, rb))
        accf_s = accf_s + wt * (jax.nn.relu(c1 * w1b + b1b)
                                + jax.nn.relu(c2 * w1b + b1b))
        c1d = jnp.broadcast_to(cs[K + t:K + t + 1, :], (D, rb))
        c2d = jnp.broadcast_to(cd[K + t:K + t + 1, :], (D, rb))
        accf_d = accf_d + wt * (jax.nn.relu(c1d * w1b + b1b)
                                + jax.nn.relu(c2d * w1b + b1b))

    w2 = w2_ref[...]
    b2b = b2b_ref[...]
    inv = 1.0 / WSUM
    so_ref[...] = (jnp.dot(w2, accf_s, preferred_element_type=jnp.float32)
                   * inv + b2b)
    do_ref[...] = (jnp.dot(w2, accf_d, preferred_element_type=jnp.float32)
                   * inv + b2b)


def kernel(src_padded_nodes_neighbor_ids, dst_padded_nodes_neighbor_ids,
           W1, b1, W2, b2):
    src = src_padded_nodes_neighbor_ids
    dst = dst_padded_nodes_neighbor_ids
    B = src.shape[0]
    f32 = jnp.float32

    srcT = src.T
    dstT = dst.T
    tlT = jnp.concatenate([src[:, TAIL0:], dst[:, TAIL0:]], axis=1).T

    ones = jnp.ones((1, B), f32)
    w1b = W1.reshape(D, 1) * ones
    b1b = b1.reshape(D, 1) * ones
    b2b = (2.0 * b2).reshape(D, 1) * ones

    Rb = 256
    grid = (B // Rb,)
    so, do = pl.pallas_call(
        _gae_kernel,
        grid=grid,
        in_specs=[
            pl.BlockSpec((L, Rb), lambda i: (0, i)),
            pl.BlockSpec((L, Rb), lambda i: (0, i)),
            pl.BlockSpec((T2, Rb), lambda i: (0, i)),
            pl.BlockSpec((D, Rb), lambda i: (0, i)),
            pl.BlockSpec((D, Rb), lambda i: (0, i)),
            pl.BlockSpec((D, D), lambda i: (0, 0)),
            pl.BlockSpec((D, Rb), lambda i: (0, i)),
        ],
        out_specs=[pl.BlockSpec((D, Rb), lambda i: (0, i)),
                   pl.BlockSpec((D, Rb), lambda i: (0, i))],
        out_shape=[jax.ShapeDtypeStruct((D, B), f32),
                   jax.ShapeDtypeStruct((D, B), f32)],
        compiler_params=pltpu.CompilerParams(
            dimension_semantics=("parallel",)),
    )(srcT, dstT, tlT, w1b, b1b, W2, b2b)
    return (so.T, do.T)
